# speedup vs baseline: 2.8876x; 2.8876x over previous
import functools

import jax
import jax.numpy as jnp
from jax import lax
from jax.experimental import pallas as pl
from jax.experimental.pallas import tpu as pltpu
from jax.experimental.pallas import tpu_sc as plsc

_NUM_CORES = 2
_NUM_SUBCORES = 16
_NUM_WORKERS = _NUM_CORES * _NUM_SUBCORES

_CHUNK = 32


@functools.cache
def _build(batch, seq_len, d_model, dtype):
    rows_per_w = seq_len // _NUM_WORKERS
    assert seq_len % _NUM_WORKERS == 0
    assert rows_per_w % _CHUNK == 0
    n_chunks = rows_per_w // _CHUNK

    mesh = plsc.VectorSubcoreMesh(
        core_axis_name="c", subcore_axis_name="s", num_cores=_NUM_CORES
    )

    @functools.partial(
        pl.kernel,
        out_type=jax.ShapeDtypeStruct((batch, seq_len, d_model), dtype),
        mesh=mesh,
        scratch_types=[
            pltpu.VMEM((_CHUNK, d_model), dtype),
            pltpu.SemaphoreType.DMA,
        ],
    )
    def broadcast_rows(table_hbm, out_hbm, buf, sem):
        wid = lax.axis_index("s") * _NUM_CORES + lax.axis_index("c")
        base = wid * rows_per_w
        for t in range(n_chunks):
            start = base + t * _CHUNK
            pltpu.async_copy(table_hbm.at[pl.ds(start, _CHUNK)], buf, sem).wait()
            for b in range(batch):
                pltpu.sync_copy(buf, out_hbm.at[b].at[pl.ds(start, _CHUNK)])

    return broadcast_rows


def kernel(inputs, pos_embedding):
    batch, seq_len = inputs.shape
    _, d_model = pos_embedding.shape
    fn = _build(batch, seq_len, d_model, pos_embedding.dtype)
    return fn(pos_embedding)


# speedup vs baseline: 2.9713x; 1.0290x over previous
import functools

import jax
import jax.numpy as jnp
from jax import lax
from jax.experimental import pallas as pl
from jax.experimental.pallas import tpu as pltpu
from jax.experimental.pallas import tpu_sc as plsc

_NUM_CORES = 2
_NUM_SUBCORES = 16
_NUM_WORKERS = _NUM_CORES * _NUM_SUBCORES

_CHUNK = 32


@functools.cache
def _build(batch, seq_len, d_model, dtype):
    rows_per_w = seq_len // _NUM_WORKERS
    assert seq_len % _NUM_WORKERS == 0
    assert rows_per_w % _CHUNK == 0
    n_chunks = rows_per_w // _CHUNK

    mesh = plsc.VectorSubcoreMesh(
        core_axis_name="c", subcore_axis_name="s", num_cores=_NUM_CORES
    )

    @functools.partial(
        pl.kernel,
        out_type=jax.ShapeDtypeStruct((batch, seq_len, d_model), dtype),
        mesh=mesh,
        scratch_types=[
            pltpu.VMEM((_CHUNK, d_model), dtype),
            pltpu.VMEM((_CHUNK, d_model), dtype),
            pltpu.SemaphoreType.DMA,
            pltpu.SemaphoreType.DMA,
            pltpu.SemaphoreType.DMA,
            pltpu.SemaphoreType.DMA,
        ],
    )
    def broadcast_rows(table_hbm, out_hbm, buf0, buf1, rsem0, rsem1, wsem0, wsem1):
        bufs = (buf0, buf1)
        rsems = (rsem0, rsem1)
        wsems = (wsem0, wsem1)
        wid = lax.axis_index("s") * _NUM_CORES + lax.axis_index("c")
        base = wid * rows_per_w

        def read(t):
            return pltpu.make_async_copy(
                table_hbm.at[pl.ds(base + t * _CHUNK, _CHUNK)],
                bufs[t % 2],
                rsems[t % 2],
            )

        def writes(t):
            return [
                pltpu.make_async_copy(
                    bufs[t % 2],
                    out_hbm.at[b].at[pl.ds(base + t * _CHUNK, _CHUNK)],
                    wsems[t % 2],
                )
                for b in range(batch)
            ]

        read(0).start()
        if n_chunks > 1:
            read(1).start()
        for t in range(n_chunks):
            read(t).wait()
            for w in writes(t):
                w.start()
            if t + 2 < n_chunks:
                for w in writes(t):
                    w.wait()
                read(t + 2).start()
        for t in (n_chunks - 2, n_chunks - 1):
            if t >= 0 and t + 2 >= n_chunks:
                for w in writes(t):
                    w.wait()

    return broadcast_rows


def kernel(inputs, pos_embedding):
    batch, seq_len = inputs.shape
    _, d_model = pos_embedding.shape
    fn = _build(batch, seq_len, d_model, pos_embedding.dtype)
    return fn(pos_embedding)
